# Initial kernel scaffold; baseline (speedup 1.0000x reference)
import jax
import jax.numpy as jnp
from jax.experimental import pallas as pl


def kernel(x, edge_index, W1, b1, W2, b2, W3, b3):
    raise NotImplementedError("write your pallas kernel here")


# speedup vs baseline: 20.5161x; 20.5161x over previous
import functools

import jax
import jax.numpy as jnp
from jax import lax
from jax.experimental import pallas as pl
from jax.experimental.pallas import tpu as pltpu
from jax.experimental.pallas import tpu_sc as plsc

N = 10000
E = 320000
D_IN = 128
H1 = 128
H2 = 64
C = 16

NC = 2
NS = 16
NW = NC * NS
EPW = E // NW
K = 80
NCH = EPW // K
NPAD = 10240
RPT = NPAD // NS
DEG_W = 8

_MESH = plsc.VectorSubcoreMesh(core_axis_name="c", subcore_axis_name="s")


@functools.partial(
    pl.kernel,
    out_type=jax.ShapeDtypeStruct((NC, NPAD, DEG_W), jnp.float32),
    mesh=_MESH,
    scratch_types=[
        pltpu.VMEM((NCH, K), jnp.int32),
        pltpu.VMEM((K, DEG_W), jnp.float32),
        pltpu.VMEM_SHARED((NPAD, DEG_W), jnp.float32),
    ],
    compiler_params=pltpu.CompilerParams(use_tc_tiling_on_sc=False),
)
def _deg(dst_hbm, zeros_hbm, ones_hbm, out_hbm, didx, onesb, acc):
    cid = lax.axis_index("c")
    sid = lax.axis_index("s")
    wid = cid * NS + sid
    pltpu.sync_copy(zeros_hbm, acc.at[pl.ds(sid * RPT, RPT)])
    pltpu.sync_copy(ones_hbm, onesb)
    pltpu.sync_copy(dst_hbm.at[wid], didx)
    plsc.subcore_barrier()

    def body(j, carry):
        pltpu.sync_copy(onesb, acc.at[didx.at[j]], add=True)
        return carry

    lax.fori_loop(0, NCH, body, 0)
    plsc.subcore_barrier()
    pltpu.sync_copy(acc.at[pl.ds(sid * RPT, RPT)],
                    out_hbm.at[cid, pl.ds(sid * RPT, RPT)])


def _make_agg(D):
    @functools.partial(
        pl.kernel,
        out_type=jax.ShapeDtypeStruct((NC, NPAD, D), jnp.float32),
        mesh=_MESH,
        scratch_types=[
            pltpu.VMEM((EPW,), jnp.int32),
            pltpu.VMEM((NCH, K), jnp.int32),
            pltpu.VMEM((K, D), jnp.float32),
            pltpu.VMEM_SHARED((NPAD, D), jnp.float32),
            pltpu.SemaphoreType.DMA,
        ],
        compiler_params=pltpu.CompilerParams(use_tc_tiling_on_sc=False),
        name=f"gcn_agg_{D}",
    )
    def agg(y_hbm, src_hbm, dst_hbm, zeros_hbm, out_hbm,
            sidx, didx, buf, acc, sem):
        cid = lax.axis_index("c")
        sid = lax.axis_index("s")
        wid = cid * NS + sid
        pltpu.sync_copy(zeros_hbm, acc.at[pl.ds(sid * RPT, RPT)])
        pltpu.sync_copy(src_hbm.at[wid], sidx)
        pltpu.sync_copy(dst_hbm.at[wid], didx)
        plsc.subcore_barrier()

        def body(j, carry):
            pltpu.async_copy(y_hbm.at[sidx.at[pl.ds(j * K, K)]], buf,
                             sem).wait()
            pltpu.sync_copy(buf, acc.at[didx.at[j]], add=True)
            return carry

        lax.fori_loop(0, NCH, body, 0)
        plsc.subcore_barrier()
        pltpu.sync_copy(acc.at[pl.ds(sid * RPT, RPT)],
                        out_hbm.at[cid, pl.ds(sid * RPT, RPT)])

    return agg


_agg128 = _make_agg(H1)
_agg64 = _make_agg(H2)
_agg16 = _make_agg(C)


BM = 1000


def _dinv_of(dp_ref):
    deg = dp_ref[0, :, 0:1] + dp_ref[1, :, 0:1] + 1.0
    return lax.rsqrt(deg)


def _mm1_body(x_ref, w_ref, dp_ref, o_ref):
    dinv = _dinv_of(dp_ref)
    o_ref[...] = jnp.dot(x_ref[...], w_ref[...],
                         preferred_element_type=jnp.float32) * dinv


def _mm1(x, W, dp):
    return pl.pallas_call(
        _mm1_body,
        grid=(N // BM,),
        in_specs=[
            pl.BlockSpec((BM, D_IN), lambda i: (i, 0)),
            pl.BlockSpec((D_IN, H1), lambda i: (0, 0)),
            pl.BlockSpec((NC, BM, DEG_W), lambda i: (0, i, 0)),
        ],
        out_specs=pl.BlockSpec((BM, H1), lambda i: (i, 0)),
        out_shape=jax.ShapeDtypeStruct((N, H1), jnp.float32),
    )(x, W, dp)


def _mmf_body(z_ref, y_ref, b_ref, w_ref, dp_ref, o_ref):
    dinv = _dinv_of(dp_ref)
    h = jnp.maximum(dinv * (z_ref[0] + z_ref[1] + y_ref[...]) + b_ref[...],
                    0.0)
    o_ref[...] = jnp.dot(h, w_ref[...],
                         preferred_element_type=jnp.float32) * dinv


def _mmf(z, y, b, W, dp):
    din, dout = W.shape
    return pl.pallas_call(
        _mmf_body,
        grid=(N // BM,),
        in_specs=[
            pl.BlockSpec((NC, BM, din), lambda i: (0, i, 0)),
            pl.BlockSpec((BM, din), lambda i: (i, 0)),
            pl.BlockSpec((1, din), lambda i: (0, 0)),
            pl.BlockSpec((din, dout), lambda i: (0, 0)),
            pl.BlockSpec((NC, BM, DEG_W), lambda i: (0, i, 0)),
        ],
        out_specs=pl.BlockSpec((BM, dout), lambda i: (i, 0)),
        out_shape=jax.ShapeDtypeStruct((N, dout), jnp.float32),
    )(z, y, b, W, dp)


def _sm_body(z_ref, y_ref, b_ref, dp_ref, o_ref):
    dinv = _dinv_of(dp_ref)
    logits = dinv * (z_ref[0] + z_ref[1] + y_ref[...]) + b_ref[...]
    m = jnp.max(logits, axis=1, keepdims=True)
    e = jnp.exp(logits - m)
    o_ref[...] = e / jnp.sum(e, axis=1, keepdims=True)


def _sm(z, y, b, dp):
    return pl.pallas_call(
        _sm_body,
        grid=(N // BM,),
        in_specs=[
            pl.BlockSpec((NC, BM, C), lambda i: (0, i, 0)),
            pl.BlockSpec((BM, C), lambda i: (i, 0)),
            pl.BlockSpec((1, C), lambda i: (0, 0)),
            pl.BlockSpec((NC, BM, DEG_W), lambda i: (0, i, 0)),
        ],
        out_specs=pl.BlockSpec((BM, C), lambda i: (i, 0)),
        out_shape=jax.ShapeDtypeStruct((N, C), jnp.float32),
    )(z, y, b, dp)


def kernel(x, edge_index, W1, b1, W2, b2, W3, b3):
    src = edge_index[0].reshape(NW, EPW)
    dst = edge_index[1].reshape(NW, NCH, K)

    dp = _deg(dst,
              jnp.zeros((RPT, DEG_W), jnp.float32),
              jnp.ones((K, DEG_W), jnp.float32))

    y1 = _mm1(x, W1, dp)
    z1 = _agg128(y1, src, dst, jnp.zeros((RPT, H1), jnp.float32))
    y2 = _mmf(z1, y1, b1.reshape(1, H1), W2, dp)
    z2 = _agg64(y2, src, dst, jnp.zeros((RPT, H2), jnp.float32))
    y3 = _mmf(z2, y2, b2.reshape(1, H2), W3, dp)
    z3 = _agg16(y3, src, dst, jnp.zeros((RPT, C), jnp.float32))
    return _sm(z3, y3, b3.reshape(1, C), dp)


# speedup vs baseline: 37.6684x; 1.8360x over previous
import functools

import jax
import jax.numpy as jnp
from jax import lax
from jax.experimental import pallas as pl
from jax.experimental.pallas import tpu as pltpu
from jax.experimental.pallas import tpu_sc as plsc

N = 10000
E = 320000
D_IN = 128
H1 = 128
H2 = 64
C = 16

NC = 2
NS = 16
NW = NC * NS
EPW = E // NW
RPT = N // NS
DEG_W = 8
DEG_K = 80

_MESH = plsc.VectorSubcoreMesh(core_axis_name="c", subcore_axis_name="s")
_SC_PARAMS = pltpu.CompilerParams(use_tc_tiling_on_sc=False)


@functools.partial(
    pl.kernel,
    out_type=jax.ShapeDtypeStruct((NC, N, DEG_W), jnp.float32),
    mesh=_MESH,
    scratch_types=[
        pltpu.VMEM((EPW // DEG_K, DEG_K), jnp.int32),
        pltpu.VMEM((DEG_K, DEG_W), jnp.float32),
        pltpu.VMEM_SHARED((N, DEG_W), jnp.float32),
    ],
    compiler_params=_SC_PARAMS,
)
def _deg(dst_hbm, zeros_hbm, ones_hbm, out_hbm, didx, onesb, acc):
    cid = lax.axis_index("c")
    sid = lax.axis_index("s")
    wid = cid * NS + sid
    pltpu.sync_copy(zeros_hbm, acc.at[pl.ds(sid * RPT, RPT)])
    pltpu.sync_copy(ones_hbm, onesb)
    pltpu.sync_copy(dst_hbm.at[wid], didx)
    plsc.subcore_barrier()

    def body(j, carry):
        pltpu.sync_copy(onesb, acc.at[didx.at[j]], add=True)
        return carry

    lax.fori_loop(0, EPW // DEG_K, body, 0)
    plsc.subcore_barrier()
    pltpu.sync_copy(acc.at[pl.ds(sid * RPT, RPT)],
                    out_hbm.at[cid, pl.ds(sid * RPT, RPT)])


def _make_agg(D, K, nbuf, depth):
    nch = EPW // K
    ngrp = nch // nbuf

    @functools.partial(
        pl.kernel,
        out_type=jax.ShapeDtypeStruct((NC, N, D), jnp.float32),
        mesh=_MESH,
        scratch_types=[
            pltpu.VMEM((EPW,), jnp.int32),
            pltpu.VMEM((nch, K), jnp.int32),
            pltpu.VMEM((nbuf, K, D), jnp.float32),
            pltpu.VMEM_SHARED((N, D), jnp.float32),
            [pltpu.SemaphoreType.DMA] * nbuf,
            [pltpu.SemaphoreType.DMA] * nbuf,
        ],
        compiler_params=_SC_PARAMS,
        name=f"gcn_agg_{D}",
    )
    def agg(y_hbm, src_hbm, dst_hbm, zeros_hbm, out_hbm,
            sidx, didx, bufs, acc, gsem, ssem):
        cid = lax.axis_index("c")
        sid = lax.axis_index("s")
        wid = cid * NS + sid
        pltpu.sync_copy(src_hbm.at[wid], sidx)
        pltpu.sync_copy(dst_hbm.at[wid], didx)
        for b in range(depth):
            pltpu.async_copy(y_hbm.at[sidx.at[pl.ds(b * K, K)]],
                             bufs.at[b], gsem[b])
        pltpu.sync_copy(zeros_hbm, acc.at[pl.ds(sid * RPT, RPT)])
        plsc.subcore_barrier()

        def body(g, carry):
            for b in range(nbuf):
                j = g * nbuf + b
                jg = j + depth
                bg = (b + depth) % nbuf

                @pl.when((jg >= nbuf) & (jg < nch))
                def _():
                    pltpu.make_async_copy(bufs.at[bg], acc.at[didx.at[0]],
                                          ssem[bg]).wait()

                @pl.when(jg < nch)
                def _():
                    pltpu.async_copy(y_hbm.at[sidx.at[pl.ds(jg * K, K)]],
                                     bufs.at[bg], gsem[bg])

                pltpu.make_async_copy(y_hbm.at[sidx.at[pl.ds(j * K, K)]],
                                      bufs.at[b], gsem[b]).wait()
                pltpu.async_copy(bufs.at[b], acc.at[didx.at[j]], ssem[b],
                                 add=True)
            return carry

        lax.fori_loop(0, ngrp, body, 0)
        for b in range(nbuf):
            pltpu.make_async_copy(bufs.at[b], acc.at[didx.at[0]],
                                  ssem[b]).wait()
        plsc.subcore_barrier()
        pltpu.sync_copy(acc.at[pl.ds(sid * RPT, RPT)],
                        out_hbm.at[cid, pl.ds(sid * RPT, RPT)])

    return agg


_agg128 = _make_agg(H1, K=40, nbuf=5, depth=3)
_agg64 = _make_agg(H2, K=80, nbuf=5, depth=3)
_agg16 = _make_agg(C, K=80, nbuf=5, depth=3)


BM = 1000


def _dinv_of(dp_ref):
    deg = dp_ref[0, :, 0:1] + dp_ref[1, :, 0:1] + 1.0
    return lax.rsqrt(deg)


def _mm1_body(x_ref, w_ref, dp_ref, o_ref):
    dinv = _dinv_of(dp_ref)
    o_ref[...] = jnp.dot(x_ref[...], w_ref[...],
                         preferred_element_type=jnp.float32) * dinv


def _mm1(x, W, dp):
    return pl.pallas_call(
        _mm1_body,
        grid=(N // BM,),
        in_specs=[
            pl.BlockSpec((BM, D_IN), lambda i: (i, 0)),
            pl.BlockSpec((D_IN, H1), lambda i: (0, 0)),
            pl.BlockSpec((NC, BM, DEG_W), lambda i: (0, i, 0)),
        ],
        out_specs=pl.BlockSpec((BM, H1), lambda i: (i, 0)),
        out_shape=jax.ShapeDtypeStruct((N, H1), jnp.float32),
    )(x, W, dp)


def _mmf_body(z_ref, y_ref, b_ref, w_ref, dp_ref, o_ref):
    dinv = _dinv_of(dp_ref)
    h = jnp.maximum(dinv * (z_ref[0] + z_ref[1] + y_ref[...]) + b_ref[...],
                    0.0)
    o_ref[...] = jnp.dot(h, w_ref[...],
                         preferred_element_type=jnp.float32) * dinv


def _mmf(z, y, b, W, dp):
    din, dout = W.shape
    return pl.pallas_call(
        _mmf_body,
        grid=(N // BM,),
        in_specs=[
            pl.BlockSpec((NC, BM, din), lambda i: (0, i, 0)),
            pl.BlockSpec((BM, din), lambda i: (i, 0)),
            pl.BlockSpec((1, din), lambda i: (0, 0)),
            pl.BlockSpec((din, dout), lambda i: (0, 0)),
            pl.BlockSpec((NC, BM, DEG_W), lambda i: (0, i, 0)),
        ],
        out_specs=pl.BlockSpec((BM, dout), lambda i: (i, 0)),
        out_shape=jax.ShapeDtypeStruct((N, dout), jnp.float32),
    )(z, y, b, W, dp)


def _sm_body(z_ref, y_ref, b_ref, dp_ref, o_ref):
    dinv = _dinv_of(dp_ref)
    logits = dinv * (z_ref[0] + z_ref[1] + y_ref[...]) + b_ref[...]
    m = jnp.max(logits, axis=1, keepdims=True)
    e = jnp.exp(logits - m)
    o_ref[...] = e / jnp.sum(e, axis=1, keepdims=True)


def _sm(z, y, b, dp):
    return pl.pallas_call(
        _sm_body,
        grid=(N // BM,),
        in_specs=[
            pl.BlockSpec((NC, BM, C), lambda i: (0, i, 0)),
            pl.BlockSpec((BM, C), lambda i: (i, 0)),
            pl.BlockSpec((1, C), lambda i: (0, 0)),
            pl.BlockSpec((NC, BM, DEG_W), lambda i: (0, i, 0)),
        ],
        out_specs=pl.BlockSpec((BM, C), lambda i: (i, 0)),
        out_shape=jax.ShapeDtypeStruct((N, C), jnp.float32),
    )(z, y, b, dp)


def kernel(x, edge_index, W1, b1, W2, b2, W3, b3):
    src = edge_index[0].reshape(NW, EPW)
    dst = edge_index[1].reshape(NW, EPW)

    dp = _deg(dst.reshape(NW, EPW // DEG_K, DEG_K),
              jnp.zeros((RPT, DEG_W), jnp.float32),
              jnp.ones((DEG_K, DEG_W), jnp.float32))

    y1 = _mm1(x, W1, dp)
    z1 = _agg128(y1, src, dst.reshape(NW, EPW // 40, 40),
                 jnp.zeros((RPT, H1), jnp.float32))
    y2 = _mmf(z1, y1, b1.reshape(1, H1), W2, dp)
    z2 = _agg64(y2, src, dst.reshape(NW, EPW // 80, 80),
                jnp.zeros((RPT, H2), jnp.float32))
    y3 = _mmf(z2, y2, b2.reshape(1, H2), W3, dp)
    z3 = _agg16(y3, src, dst.reshape(NW, EPW // 80, 80),
                jnp.zeros((RPT, C), jnp.float32))
    return _sm(z3, y3, b3.reshape(1, C), dp)


# speedup vs baseline: 40.4018x; 1.0726x over previous
import functools

import jax
import jax.numpy as jnp
from jax import lax
from jax.experimental import pallas as pl
from jax.experimental.pallas import tpu as pltpu
from jax.experimental.pallas import tpu_sc as plsc

N = 10000
E = 320000
D_IN = 128
H1 = 128
H2 = 64
C = 16

NC = 2
NS = 16
NW = NC * NS
EPW = E // NW
RPT = N // NS
DEG_W = 8
DEG_K = 80

_MESH = plsc.VectorSubcoreMesh(core_axis_name="c", subcore_axis_name="s")
_SC_PARAMS = pltpu.CompilerParams(use_tc_tiling_on_sc=False)


@functools.partial(
    pl.kernel,
    out_type=jax.ShapeDtypeStruct((NC, N, DEG_W), jnp.float32),
    mesh=_MESH,
    scratch_types=[
        pltpu.VMEM((EPW // DEG_K, DEG_K), jnp.int32),
        pltpu.VMEM((DEG_K, DEG_W), jnp.float32),
        pltpu.VMEM_SHARED((N, DEG_W), jnp.float32),
        pltpu.SemaphoreType.DMA,
    ],
    compiler_params=_SC_PARAMS,
)
def _deg(dst_hbm, zeros_hbm, ones_hbm, out_hbm, didx, onesb, acc, dsem):
    cid = lax.axis_index("c")
    sid = lax.axis_index("s")
    wid = cid * NS + sid
    pltpu.sync_copy(zeros_hbm, acc.at[pl.ds(sid * RPT, RPT)])
    pltpu.sync_copy(ones_hbm, onesb)
    pltpu.sync_copy(dst_hbm.at[wid], didx)
    plsc.subcore_barrier()

    def body(j, carry):
        pltpu.async_copy(onesb, acc.at[didx.at[j]], dsem, add=True)
        return carry

    lax.fori_loop(0, EPW // DEG_K, body, 0)

    def drain(j, carry):
        pltpu.make_async_copy(onesb, acc.at[didx.at[0]], dsem).wait()
        return carry

    lax.fori_loop(0, EPW // DEG_K, drain, 0)
    plsc.subcore_barrier()
    pltpu.sync_copy(acc.at[pl.ds(sid * RPT, RPT)],
                    out_hbm.at[cid, pl.ds(sid * RPT, RPT)])


def _make_agg(D, K, nbuf, depth):
    nch = EPW // K
    ngrp = (nch + nbuf - 1) // nbuf

    @functools.partial(
        pl.kernel,
        out_type=jax.ShapeDtypeStruct((NC, N, D), jnp.float32),
        mesh=_MESH,
        scratch_types=[
            pltpu.VMEM((EPW,), jnp.int32),
            pltpu.VMEM((nch, K), jnp.int32),
            pltpu.VMEM((nbuf, K, D), jnp.float32),
            pltpu.VMEM_SHARED((N, D), jnp.float32),
            [pltpu.SemaphoreType.DMA] * nbuf,
            [pltpu.SemaphoreType.DMA] * nbuf,
        ],
        compiler_params=_SC_PARAMS,
        name=f"gcn_agg_{D}",
    )
    def agg(y_hbm, src_hbm, dst_hbm, zeros_hbm, out_hbm,
            sidx, didx, bufs, acc, gsem, ssem):
        cid = lax.axis_index("c")
        sid = lax.axis_index("s")
        wid = cid * NS + sid
        pltpu.sync_copy(src_hbm.at[wid], sidx)
        pltpu.sync_copy(dst_hbm.at[wid], didx)
        for b in range(depth):
            pltpu.async_copy(y_hbm.at[sidx.at[pl.ds(b * K, K)]],
                             bufs.at[b], gsem[b])
        pltpu.sync_copy(zeros_hbm, acc.at[pl.ds(sid * RPT, RPT)])
        plsc.subcore_barrier()

        def body(g, carry):
            for b in range(nbuf):
                j = g * nbuf + b
                jg = j + depth
                bg = (b + depth) % nbuf

                @pl.when((jg >= nbuf) & (jg < nch))
                def _():
                    pltpu.make_async_copy(bufs.at[bg], acc.at[didx.at[0]],
                                          ssem[bg]).wait()

                @pl.when(jg < nch)
                def _():
                    pltpu.async_copy(y_hbm.at[sidx.at[pl.ds(jg * K, K)]],
                                     bufs.at[bg], gsem[bg])

                @pl.when(j < nch)
                def _():
                    pltpu.make_async_copy(
                        y_hbm.at[sidx.at[pl.ds(j * K, K)]], bufs.at[b],
                        gsem[b]).wait()
                    pltpu.async_copy(bufs.at[b], acc.at[didx.at[j]], ssem[b],
                                     add=True)
            return carry

        lax.fori_loop(0, ngrp, body, 0)
        for b in range(nbuf):
            pltpu.make_async_copy(bufs.at[b], acc.at[didx.at[0]],
                                  ssem[b]).wait()
        plsc.subcore_barrier()
        pltpu.sync_copy(acc.at[pl.ds(sid * RPT, RPT)],
                        out_hbm.at[cid, pl.ds(sid * RPT, RPT)])

    return agg


_agg128 = _make_agg(H1, K=80, nbuf=3, depth=2)
_agg64 = _make_agg(H2, K=80, nbuf=8, depth=5)
_agg16 = _make_agg(C, K=80, nbuf=10, depth=6)


BM = 1000


def _dinv_of(dp_ref):
    deg = dp_ref[0, :, 0:1] + dp_ref[1, :, 0:1] + 1.0
    return lax.rsqrt(deg)


def _mm1_body(x_ref, w_ref, dp_ref, o_ref):
    dinv = _dinv_of(dp_ref)
    o_ref[...] = jnp.dot(x_ref[...], w_ref[...],
                         preferred_element_type=jnp.float32) * dinv


def _mm1(x, W, dp):
    return pl.pallas_call(
        _mm1_body,
        grid=(N // BM,),
        in_specs=[
            pl.BlockSpec((BM, D_IN), lambda i: (i, 0)),
            pl.BlockSpec((D_IN, H1), lambda i: (0, 0)),
            pl.BlockSpec((NC, BM, DEG_W), lambda i: (0, i, 0)),
        ],
        out_specs=pl.BlockSpec((BM, H1), lambda i: (i, 0)),
        out_shape=jax.ShapeDtypeStruct((N, H1), jnp.float32),
    )(x, W, dp)


def _mmf_body(z_ref, y_ref, b_ref, w_ref, dp_ref, o_ref):
    dinv = _dinv_of(dp_ref)
    h = jnp.maximum(dinv * (z_ref[0] + z_ref[1] + y_ref[...]) + b_ref[...],
                    0.0)
    o_ref[...] = jnp.dot(h, w_ref[...],
                         preferred_element_type=jnp.float32) * dinv


def _mmf(z, y, b, W, dp):
    din, dout = W.shape
    return pl.pallas_call(
        _mmf_body,
        grid=(N // BM,),
        in_specs=[
            pl.BlockSpec((NC, BM, din), lambda i: (0, i, 0)),
            pl.BlockSpec((BM, din), lambda i: (i, 0)),
            pl.BlockSpec((1, din), lambda i: (0, 0)),
            pl.BlockSpec((din, dout), lambda i: (0, 0)),
            pl.BlockSpec((NC, BM, DEG_W), lambda i: (0, i, 0)),
        ],
        out_specs=pl.BlockSpec((BM, dout), lambda i: (i, 0)),
        out_shape=jax.ShapeDtypeStruct((N, dout), jnp.float32),
    )(z, y, b, W, dp)


def _sm_body(z_ref, y_ref, b_ref, dp_ref, o_ref):
    dinv = _dinv_of(dp_ref)
    logits = dinv * (z_ref[0] + z_ref[1] + y_ref[...]) + b_ref[...]
    m = jnp.max(logits, axis=1, keepdims=True)
    e = jnp.exp(logits - m)
    o_ref[...] = e / jnp.sum(e, axis=1, keepdims=True)


def _sm(z, y, b, dp):
    return pl.pallas_call(
        _sm_body,
        grid=(N // BM,),
        in_specs=[
            pl.BlockSpec((NC, BM, C), lambda i: (0, i, 0)),
            pl.BlockSpec((BM, C), lambda i: (i, 0)),
            pl.BlockSpec((1, C), lambda i: (0, 0)),
            pl.BlockSpec((NC, BM, DEG_W), lambda i: (0, i, 0)),
        ],
        out_specs=pl.BlockSpec((BM, C), lambda i: (i, 0)),
        out_shape=jax.ShapeDtypeStruct((N, C), jnp.float32),
    )(z, y, b, dp)


def kernel(x, edge_index, W1, b1, W2, b2, W3, b3):
    src = edge_index[0].reshape(NW, EPW)
    dst = edge_index[1].reshape(NW, EPW)

    dp = _deg(dst.reshape(NW, EPW // DEG_K, DEG_K),
              jnp.zeros((RPT, DEG_W), jnp.float32),
              jnp.ones((DEG_K, DEG_W), jnp.float32))

    y1 = _mm1(x, W1, dp)
    z1 = _agg128(y1, src, dst.reshape(NW, EPW // 80, 80),
                 jnp.zeros((RPT, H1), jnp.float32))
    y2 = _mmf(z1, y1, b1.reshape(1, H1), W2, dp)
    z2 = _agg64(y2, src, dst.reshape(NW, EPW // 80, 80),
                jnp.zeros((RPT, H2), jnp.float32))
    y3 = _mmf(z2, y2, b2.reshape(1, H2), W3, dp)
    z3 = _agg16(y3, src, dst.reshape(NW, EPW // 80, 80),
                jnp.zeros((RPT, C), jnp.float32))
    return _sm(z3, y3, b3.reshape(1, C), dp)


# speedup vs baseline: 43.2075x; 1.0694x over previous
import functools

import jax
import jax.numpy as jnp
from jax import lax
from jax.experimental import pallas as pl
from jax.experimental.pallas import tpu as pltpu
from jax.experimental.pallas import tpu_sc as plsc

N = 10000
E = 320000
D_IN = 128
H1 = 128
H2 = 64
C = 16

NC = 2
NS = 16
NW = NC * NS
EPW = E // NW
RPT = N // NS
DEG_W = 8
DEG_K = 80

_MESH = plsc.VectorSubcoreMesh(core_axis_name="c", subcore_axis_name="s")
_SC_PARAMS = pltpu.CompilerParams(use_tc_tiling_on_sc=False)


@functools.partial(
    pl.kernel,
    out_type=jax.ShapeDtypeStruct((NC, N, DEG_W), jnp.float32),
    mesh=_MESH,
    scratch_types=[
        pltpu.VMEM((EPW // DEG_K, DEG_K), jnp.int32),
        pltpu.VMEM((DEG_K, DEG_W), jnp.float32),
        pltpu.VMEM_SHARED((N, DEG_W), jnp.float32),
        pltpu.SemaphoreType.DMA,
    ],
    compiler_params=_SC_PARAMS,
)
def _deg(dst_hbm, zeros_hbm, ones_hbm, out_hbm, didx, onesb, acc, dsem):
    cid = lax.axis_index("c")
    sid = lax.axis_index("s")
    wid = cid * NS + sid
    pltpu.sync_copy(zeros_hbm, acc.at[pl.ds(sid * RPT, RPT)])
    pltpu.sync_copy(ones_hbm, onesb)
    pltpu.sync_copy(dst_hbm.at[wid], didx)
    plsc.subcore_barrier()

    def body(j, carry):
        pltpu.async_copy(onesb, acc.at[didx.at[j]], dsem, add=True)
        return carry

    lax.fori_loop(0, EPW // DEG_K, body, 0)

    def drain(j, carry):
        pltpu.make_async_copy(onesb, acc.at[didx.at[0]], dsem).wait()
        return carry

    lax.fori_loop(0, EPW // DEG_K, drain, 0)
    plsc.subcore_barrier()
    pltpu.sync_copy(acc.at[pl.ds(sid * RPT, RPT)],
                    out_hbm.at[cid, pl.ds(sid * RPT, RPT)])


def _make_agg(D, K, nbuf, depth):
    nch = EPW // K
    ngrp = (nch + nbuf - 1) // nbuf

    @functools.partial(
        pl.kernel,
        out_type=jax.ShapeDtypeStruct((NC, N, D), jnp.float32),
        mesh=_MESH,
        scratch_types=[
            pltpu.VMEM((EPW,), jnp.int32),
            pltpu.VMEM((nch, K), jnp.int32),
            pltpu.VMEM((nbuf, K, D), jnp.float32),
            pltpu.VMEM_SHARED((N, D), jnp.float32),
            [pltpu.SemaphoreType.DMA] * nbuf,
            [pltpu.SemaphoreType.DMA] * nbuf,
        ],
        compiler_params=_SC_PARAMS,
        name=f"gcn_agg_{D}",
    )
    def agg(y_hbm, src_hbm, dst_hbm, zeros_hbm, out_hbm,
            sidx, didx, bufs, acc, gsem, ssem):
        cid = lax.axis_index("c")
        sid = lax.axis_index("s")
        wid = cid * NS + sid
        pltpu.sync_copy(src_hbm.at[wid], sidx)
        pltpu.sync_copy(dst_hbm.at[wid], didx)
        for b in range(depth):
            pltpu.async_copy(y_hbm.at[sidx.at[pl.ds(b * K, K)]],
                             bufs.at[b], gsem[b])
        pltpu.sync_copy(zeros_hbm, acc.at[pl.ds(sid * RPT, RPT)])
        plsc.subcore_barrier()

        def body(g, carry):
            for b in range(nbuf):
                j = g * nbuf + b
                jg = j + depth
                bg = (b + depth) % nbuf

                @pl.when((jg >= nbuf) & (jg < nch))
                def _():
                    pltpu.make_async_copy(bufs.at[bg], acc.at[didx.at[0]],
                                          ssem[bg]).wait()

                @pl.when(jg < nch)
                def _():
                    pltpu.async_copy(y_hbm.at[sidx.at[pl.ds(jg * K, K)]],
                                     bufs.at[bg], gsem[bg])

                @pl.when(j < nch)
                def _():
                    pltpu.make_async_copy(
                        y_hbm.at[sidx.at[pl.ds(j * K, K)]], bufs.at[b],
                        gsem[b]).wait()
                    pltpu.async_copy(bufs.at[b], acc.at[didx.at[j]], ssem[b],
                                     add=True)
            return carry

        lax.fori_loop(0, ngrp, body, 0)
        for b in range(nbuf):
            pltpu.make_async_copy(bufs.at[b], acc.at[didx.at[0]],
                                  ssem[b]).wait()
        plsc.subcore_barrier()
        pltpu.sync_copy(acc.at[pl.ds(sid * RPT, RPT)],
                        out_hbm.at[cid, pl.ds(sid * RPT, RPT)])

    return agg


_agg128 = _make_agg(H1, K=80, nbuf=3, depth=2)
_agg64 = _make_agg(H2, K=80, nbuf=8, depth=5)
_agg16 = _make_agg(C, K=80, nbuf=10, depth=6)


BM = 2000


def _esplit_body(e_ref, s_ref, d_ref):
    s_ref[...] = e_ref[0:1, :].reshape(E // 128, 128)
    d_ref[...] = e_ref[1:2, :].reshape(E // 128, 128)


def _esplit(edge_index):
    return pl.pallas_call(
        _esplit_body,
        grid=(1,),
        in_specs=[pl.BlockSpec((2, E), lambda i: (0, 0))],
        out_specs=[
            pl.BlockSpec((E // 128, 128), lambda i: (0, 0)),
            pl.BlockSpec((E // 128, 128), lambda i: (0, 0)),
        ],
        out_shape=[
            jax.ShapeDtypeStruct((E // 128, 128), jnp.int32),
            jax.ShapeDtypeStruct((E // 128, 128), jnp.int32),
        ],
    )(edge_index)


def _dinv_of(dp_ref):
    deg = dp_ref[0, :, 0:1] + dp_ref[1, :, 0:1] + 1.0
    return lax.rsqrt(deg)


def _mm1_body(x_ref, w_ref, dp_ref, o_ref):
    dinv = _dinv_of(dp_ref)
    o_ref[...] = jnp.dot(x_ref[...], w_ref[...],
                         preferred_element_type=jnp.float32) * dinv


def _mm1(x, W, dp):
    return pl.pallas_call(
        _mm1_body,
        grid=(N // BM,),
        in_specs=[
            pl.BlockSpec((BM, D_IN), lambda i: (i, 0)),
            pl.BlockSpec((D_IN, H1), lambda i: (0, 0)),
            pl.BlockSpec((NC, BM, DEG_W), lambda i: (0, i, 0)),
        ],
        out_specs=pl.BlockSpec((BM, H1), lambda i: (i, 0)),
        out_shape=jax.ShapeDtypeStruct((N, H1), jnp.float32),
    )(x, W, dp)


def _mmf_body(z_ref, y_ref, b_ref, w_ref, dp_ref, o_ref):
    dinv = _dinv_of(dp_ref)
    h = jnp.maximum(dinv * (z_ref[0] + z_ref[1] + y_ref[...]) + b_ref[...],
                    0.0)
    o_ref[...] = jnp.dot(h, w_ref[...],
                         preferred_element_type=jnp.float32) * dinv


def _mmf(z, y, b, W, dp):
    din, dout = W.shape
    return pl.pallas_call(
        _mmf_body,
        grid=(N // BM,),
        in_specs=[
            pl.BlockSpec((NC, BM, din), lambda i: (0, i, 0)),
            pl.BlockSpec((BM, din), lambda i: (i, 0)),
            pl.BlockSpec((1, din), lambda i: (0, 0)),
            pl.BlockSpec((din, dout), lambda i: (0, 0)),
            pl.BlockSpec((NC, BM, DEG_W), lambda i: (0, i, 0)),
        ],
        out_specs=pl.BlockSpec((BM, dout), lambda i: (i, 0)),
        out_shape=jax.ShapeDtypeStruct((N, dout), jnp.float32),
    )(z, y, b, W, dp)


def _sm_body(z_ref, y_ref, b_ref, dp_ref, o_ref):
    dinv = _dinv_of(dp_ref)
    logits = dinv * (z_ref[0] + z_ref[1] + y_ref[...]) + b_ref[...]
    m = jnp.max(logits, axis=1, keepdims=True)
    e = jnp.exp(logits - m)
    o_ref[...] = e / jnp.sum(e, axis=1, keepdims=True)


def _sm(z, y, b, dp):
    return pl.pallas_call(
        _sm_body,
        grid=(N // BM,),
        in_specs=[
            pl.BlockSpec((NC, BM, C), lambda i: (0, i, 0)),
            pl.BlockSpec((BM, C), lambda i: (i, 0)),
            pl.BlockSpec((1, C), lambda i: (0, 0)),
            pl.BlockSpec((NC, BM, DEG_W), lambda i: (0, i, 0)),
        ],
        out_specs=pl.BlockSpec((BM, C), lambda i: (i, 0)),
        out_shape=jax.ShapeDtypeStruct((N, C), jnp.float32),
    )(z, y, b, dp)


def kernel(x, edge_index, W1, b1, W2, b2, W3, b3):
    srcp, dstp = _esplit(edge_index)
    src = srcp.reshape(NW, EPW)
    dst = dstp.reshape(NW, EPW)

    dp = _deg(dst.reshape(NW, EPW // DEG_K, DEG_K),
              jnp.zeros((RPT, DEG_W), jnp.float32),
              jnp.ones((DEG_K, DEG_W), jnp.float32))

    y1 = _mm1(x, W1, dp)
    z1 = _agg128(y1, src, dst.reshape(NW, EPW // 80, 80),
                 jnp.zeros((RPT, H1), jnp.float32))
    y2 = _mmf(z1, y1, b1.reshape(1, H1), W2, dp)
    z2 = _agg64(y2, src, dst.reshape(NW, EPW // 80, 80),
                jnp.zeros((RPT, H2), jnp.float32))
    y3 = _mmf(z2, y2, b2.reshape(1, H2), W3, dp)
    z3 = _agg16(y3, src, dst.reshape(NW, EPW // 80, 80),
                jnp.zeros((RPT, C), jnp.float32))
    return _sm(z3, y3, b3.reshape(1, C), dp)


# speedup vs baseline: 45.2744x; 1.0478x over previous
import functools

import jax
import jax.numpy as jnp
from jax import lax
from jax.experimental import pallas as pl
from jax.experimental.pallas import tpu as pltpu
from jax.experimental.pallas import tpu_sc as plsc

N = 10000
E = 320000
D_IN = 128
H1 = 128
H2 = 64
C = 16

NC = 2
NS = 16
NW = NC * NS
EPW = E // NW
RPT = N // NS
DEG_W = 8
DEG_K = 80

_MESH = plsc.VectorSubcoreMesh(core_axis_name="c", subcore_axis_name="s")
_SC_PARAMS = pltpu.CompilerParams(use_tc_tiling_on_sc=False)


@functools.partial(
    pl.kernel,
    out_type=jax.ShapeDtypeStruct((NC, N, DEG_W), jnp.float32),
    mesh=_MESH,
    scratch_types=[
        pltpu.VMEM((EPW // DEG_K, DEG_K), jnp.int32),
        pltpu.VMEM((DEG_K, DEG_W), jnp.float32),
        pltpu.VMEM_SHARED((N, DEG_W), jnp.float32),
        pltpu.SemaphoreType.DMA,
    ],
    compiler_params=_SC_PARAMS,
)
def _deg(dst_hbm, zeros_hbm, ones_hbm, out_hbm, didx, onesb, acc, dsem):
    cid = lax.axis_index("c")
    sid = lax.axis_index("s")
    wid = cid * NS + sid
    pltpu.sync_copy(zeros_hbm, acc.at[pl.ds(sid * RPT, RPT)])
    pltpu.sync_copy(ones_hbm, onesb)
    pltpu.sync_copy(dst_hbm.at[wid], didx)
    plsc.subcore_barrier()

    def body(j, carry):
        pltpu.async_copy(onesb, acc.at[didx.at[j]], dsem, add=True)
        return carry

    lax.fori_loop(0, EPW // DEG_K, body, 0)

    def drain(j, carry):
        pltpu.make_async_copy(onesb, acc.at[didx.at[0]], dsem).wait()
        return carry

    lax.fori_loop(0, EPW // DEG_K, drain, 0)
    plsc.subcore_barrier()
    pltpu.sync_copy(acc.at[pl.ds(sid * RPT, RPT)],
                    out_hbm.at[cid, pl.ds(sid * RPT, RPT)])


def _make_agg(D, K, nbuf, depth):
    nch = EPW // K
    ngrp = (nch + nbuf - 1) // nbuf

    @functools.partial(
        pl.kernel,
        out_type=jax.ShapeDtypeStruct((NC, N, 128), jnp.float32),
        mesh=_MESH,
        scratch_types=[
            pltpu.VMEM((EPW,), jnp.int32),
            pltpu.VMEM((nch, K), jnp.int32),
            pltpu.VMEM((nbuf, K, D), jnp.float32),
            pltpu.VMEM_SHARED((N, D), jnp.float32),
            [pltpu.SemaphoreType.DMA] * nbuf,
            [pltpu.SemaphoreType.DMA] * nbuf,
        ],
        compiler_params=_SC_PARAMS,
        name=f"gcn_agg_{D}",
    )
    def agg(y_hbm, src_hbm, dst_hbm, zeros_hbm, out_hbm,
            sidx, didx, bufs, acc, gsem, ssem):
        cid = lax.axis_index("c")
        sid = lax.axis_index("s")
        wid = cid * NS + sid
        pltpu.sync_copy(src_hbm.at[wid], sidx)
        pltpu.sync_copy(dst_hbm.at[wid], didx)
        for b in range(depth):
            pltpu.async_copy(y_hbm.at[sidx.at[pl.ds(b * K, K)]],
                             bufs.at[b], gsem[b])
        pltpu.sync_copy(zeros_hbm, acc.at[pl.ds(sid * RPT, RPT)])
        plsc.subcore_barrier()

        def body(g, carry):
            for b in range(nbuf):
                j = g * nbuf + b
                jg = j + depth
                bg = (b + depth) % nbuf

                @pl.when((jg >= nbuf) & (jg < nch))
                def _():
                    pltpu.make_async_copy(bufs.at[bg], acc.at[didx.at[0]],
                                          ssem[bg]).wait()

                @pl.when(jg < nch)
                def _():
                    pltpu.async_copy(y_hbm.at[sidx.at[pl.ds(jg * K, K)]],
                                     bufs.at[bg], gsem[bg])

                @pl.when(j < nch)
                def _():
                    pltpu.make_async_copy(
                        y_hbm.at[sidx.at[pl.ds(j * K, K)]], bufs.at[b],
                        gsem[b]).wait()
                    pltpu.async_copy(bufs.at[b], acc.at[didx.at[j]], ssem[b],
                                     add=True)
            return carry

        lax.fori_loop(0, ngrp, body, 0)
        for b in range(nbuf):
            pltpu.make_async_copy(bufs.at[b], acc.at[didx.at[0]],
                                  ssem[b]).wait()
        plsc.subcore_barrier()
        pltpu.sync_copy(acc.at[pl.ds(sid * RPT, RPT)],
                        out_hbm.at[cid, pl.ds(sid * RPT, RPT), pl.ds(0, D)])

    return agg


_agg128 = _make_agg(H1, K=80, nbuf=3, depth=2)
_agg64 = _make_agg(H2, K=80, nbuf=8, depth=5)
_agg16 = _make_agg(C, K=80, nbuf=10, depth=6)


BM = 2000


def _esplit_body(e_ref, s_ref, d_ref):
    s_ref[...] = e_ref[0:1, :].reshape(E // 128, 128)
    d_ref[...] = e_ref[1:2, :].reshape(E // 128, 128)


def _esplit(edge_index):
    return pl.pallas_call(
        _esplit_body,
        grid=(1,),
        in_specs=[pl.BlockSpec((2, E), lambda i: (0, 0))],
        out_specs=[
            pl.BlockSpec((E // 128, 128), lambda i: (0, 0)),
            pl.BlockSpec((E // 128, 128), lambda i: (0, 0)),
        ],
        out_shape=[
            jax.ShapeDtypeStruct((E // 128, 128), jnp.int32),
            jax.ShapeDtypeStruct((E // 128, 128), jnp.int32),
        ],
    )(edge_index)


def _dinv_of(dp_ref):
    deg = dp_ref[0, :, 0:1] + dp_ref[1, :, 0:1] + 1.0
    return lax.rsqrt(deg)


def _mm1_body(x_ref, w_ref, dp_ref, o_ref):
    dinv = _dinv_of(dp_ref)
    o_ref[...] = jnp.dot(x_ref[...], w_ref[...],
                         preferred_element_type=jnp.float32) * dinv


def _mm1(x, W, dp):
    return pl.pallas_call(
        _mm1_body,
        grid=(N // BM,),
        in_specs=[
            pl.BlockSpec((BM, D_IN), lambda i: (i, 0)),
            pl.BlockSpec((D_IN, H1), lambda i: (0, 0)),
            pl.BlockSpec((NC, BM, DEG_W), lambda i: (0, i, 0)),
        ],
        out_specs=pl.BlockSpec((BM, H1), lambda i: (i, 0)),
        out_shape=jax.ShapeDtypeStruct((N, H1), jnp.float32),
    )(x, W, dp)


def _make_mmf_body(din, dout, pack_out):
    def _mmf_body(z_ref, y_ref, b_ref, w_ref, dp_ref, o_ref, op_ref=None):
        dinv = _dinv_of(dp_ref)
        h = jnp.maximum(
            dinv * (z_ref[0, :, 0:din] + z_ref[1, :, 0:din] + y_ref[...])
            + b_ref[...], 0.0)
        o = jnp.dot(h, w_ref[...], preferred_element_type=jnp.float32) * dinv
        o_ref[...] = o
    return _mmf_body


def _mmf(z, y, b, W, dp, pack_out):
    din, dout = W.shape
    out_shape = [jax.ShapeDtypeStruct((N, dout), jnp.float32)]
    out_specs = [pl.BlockSpec((BM, dout), lambda i: (i, 0))]
    if pack_out:
        out_shape.append(
            jax.ShapeDtypeStruct((N * dout // 128, 128), jnp.float32))
        out_specs.append(
            pl.BlockSpec((BM * dout // 128, 128), lambda i: (i, 0)))
    return pl.pallas_call(
        _make_mmf_body(din, dout, pack_out),
        grid=(N // BM,),
        in_specs=[
            pl.BlockSpec((NC, BM, 128), lambda i: (0, i, 0)),
            pl.BlockSpec((BM, din), lambda i: (i, 0)),
            pl.BlockSpec((1, din), lambda i: (0, 0)),
            pl.BlockSpec((din, dout), lambda i: (0, 0)),
            pl.BlockSpec((NC, BM, DEG_W), lambda i: (0, i, 0)),
        ],
        out_specs=out_specs,
        out_shape=out_shape,
    )(z, y, b, W, dp)


def _sm_body(z_ref, y_ref, b_ref, dp_ref, o_ref):
    dinv = _dinv_of(dp_ref)
    logits = (dinv * (z_ref[0, :, 0:C] + z_ref[1, :, 0:C] + y_ref[...])
              + b_ref[...])
    m = jnp.max(logits, axis=1, keepdims=True)
    e = jnp.exp(logits - m)
    o_ref[...] = e / jnp.sum(e, axis=1, keepdims=True)


def _sm(z, y, b, dp):
    return pl.pallas_call(
        _sm_body,
        grid=(N // BM,),
        in_specs=[
            pl.BlockSpec((NC, BM, 128), lambda i: (0, i, 0)),
            pl.BlockSpec((BM, C), lambda i: (i, 0)),
            pl.BlockSpec((1, C), lambda i: (0, 0)),
            pl.BlockSpec((NC, BM, DEG_W), lambda i: (0, i, 0)),
        ],
        out_specs=pl.BlockSpec((BM, C), lambda i: (i, 0)),
        out_shape=jax.ShapeDtypeStruct((N, C), jnp.float32),
    )(z, y, b, dp)


def kernel(x, edge_index, W1, b1, W2, b2, W3, b3):
    srcp, dstp = _esplit(edge_index)
    src = srcp.reshape(NW, EPW)
    dst = dstp.reshape(NW, EPW)

    dp = _deg(dst.reshape(NW, EPW // DEG_K, DEG_K),
              jnp.zeros((RPT, DEG_W), jnp.float32),
              jnp.ones((DEG_K, DEG_W), jnp.float32))

    y1 = _mm1(x, W1, dp)
    z1 = _agg128(y1, src, dst.reshape(NW, EPW // 80, 80),
                 jnp.zeros((RPT, H1), jnp.float32))
    y2, = _mmf(z1, y1, b1.reshape(1, H1), W2, dp, pack_out=False)
    z2 = _agg64(y2, src, dst.reshape(NW, EPW // 80, 80),
                jnp.zeros((RPT, H2), jnp.float32))
    y3, = _mmf(z2, y2, b2.reshape(1, H2), W3, dp, pack_out=False)
    z3 = _agg16(y3, src, dst.reshape(NW, EPW // 80, 80),
                jnp.zeros((RPT, C), jnp.float32))
    return _sm(z3, y3, b3.reshape(1, C), dp)


# speedup vs baseline: 45.5110x; 1.0052x over previous
import functools

import jax
import jax.numpy as jnp
from jax import lax
from jax.experimental import pallas as pl
from jax.experimental.pallas import tpu as pltpu
from jax.experimental.pallas import tpu_sc as plsc

N = 10000
E = 320000
D_IN = 128
H1 = 128
H2 = 64
C = 16

NC = 2
NS = 16
NW = NC * NS
EPW = E // NW
RPT = N // NS
DEG_W = 8
DEG_K = 80

_MESH = plsc.VectorSubcoreMesh(core_axis_name="c", subcore_axis_name="s")
_SC_PARAMS = pltpu.CompilerParams(use_tc_tiling_on_sc=False)


@functools.partial(
    pl.kernel,
    out_type=jax.ShapeDtypeStruct((NC, N, DEG_W), jnp.float32),
    mesh=_MESH,
    scratch_types=[
        pltpu.VMEM((EPW // DEG_K, DEG_K), jnp.int32),
        pltpu.VMEM((DEG_K, DEG_W), jnp.float32),
        pltpu.VMEM_SHARED((N, DEG_W), jnp.float32),
        pltpu.SemaphoreType.DMA,
    ],
    compiler_params=_SC_PARAMS,
)
def _deg(dst_hbm, zeros_hbm, ones_hbm, out_hbm, didx, onesb, acc, dsem):
    cid = lax.axis_index("c")
    sid = lax.axis_index("s")
    wid = cid * NS + sid
    pltpu.sync_copy(zeros_hbm, acc.at[pl.ds(sid * RPT, RPT)])
    pltpu.sync_copy(ones_hbm, onesb)
    pltpu.sync_copy(dst_hbm.at[wid], didx)
    plsc.subcore_barrier()

    def body(j, carry):
        pltpu.async_copy(onesb, acc.at[didx.at[j]], dsem, add=True)
        return carry

    lax.fori_loop(0, EPW // DEG_K, body, 0)

    def drain(j, carry):
        pltpu.make_async_copy(onesb, acc.at[didx.at[0]], dsem).wait()
        return carry

    lax.fori_loop(0, EPW // DEG_K, drain, 0)
    plsc.subcore_barrier()
    pltpu.sync_copy(acc.at[pl.ds(sid * RPT, RPT)],
                    out_hbm.at[cid, pl.ds(sid * RPT, RPT)])


def _make_agg(D, K, nbuf, depth):
    nch = EPW // K
    ngrp = (nch + nbuf - 1) // nbuf

    @functools.partial(
        pl.kernel,
        out_type=jax.ShapeDtypeStruct((NC, N, 128), jnp.float32),
        mesh=_MESH,
        scratch_types=[
            pltpu.VMEM((EPW,), jnp.int32),
            pltpu.VMEM((nch, K), jnp.int32),
            pltpu.VMEM((nbuf, K, D), jnp.float32),
            pltpu.VMEM_SHARED((N, D), jnp.float32),
            [pltpu.SemaphoreType.DMA] * nbuf,
            [pltpu.SemaphoreType.DMA] * nbuf,
        ],
        compiler_params=_SC_PARAMS,
        name=f"gcn_agg_{D}",
    )
    def agg(y_hbm, src_hbm, dst_hbm, zeros_hbm, out_hbm,
            sidx, didx, bufs, acc, gsem, ssem):
        cid = lax.axis_index("c")
        sid = lax.axis_index("s")
        wid = cid * NS + sid
        pltpu.sync_copy(src_hbm.at[wid], sidx)
        pltpu.sync_copy(dst_hbm.at[wid], didx)
        for b in range(depth):
            pltpu.async_copy(y_hbm.at[sidx.at[pl.ds(b * K, K)]],
                             bufs.at[b], gsem[b])
        pltpu.sync_copy(zeros_hbm, acc.at[pl.ds(sid * RPT, RPT)])
        plsc.subcore_barrier()

        def body(g, carry):
            for b in range(nbuf):
                j = g * nbuf + b
                jg = j + depth
                bg = (b + depth) % nbuf

                @pl.when((jg >= nbuf) & (jg < nch))
                def _():
                    pltpu.make_async_copy(bufs.at[bg], acc.at[didx.at[0]],
                                          ssem[bg]).wait()

                @pl.when(jg < nch)
                def _():
                    pltpu.async_copy(y_hbm.at[sidx.at[pl.ds(jg * K, K)]],
                                     bufs.at[bg], gsem[bg])

                @pl.when(j < nch)
                def _():
                    pltpu.make_async_copy(
                        y_hbm.at[sidx.at[pl.ds(j * K, K)]], bufs.at[b],
                        gsem[b]).wait()
                    pltpu.async_copy(bufs.at[b], acc.at[didx.at[j]], ssem[b],
                                     add=True)
            return carry

        lax.fori_loop(0, ngrp, body, 0)
        for b in range(nbuf):
            pltpu.make_async_copy(bufs.at[b], acc.at[didx.at[0]],
                                  ssem[b]).wait()
        plsc.subcore_barrier()
        pltpu.sync_copy(acc.at[pl.ds(sid * RPT, RPT)],
                        out_hbm.at[cid, pl.ds(sid * RPT, RPT), pl.ds(0, D)])

    return agg


_agg128 = _make_agg(H1, K=80, nbuf=3, depth=2)

CH_TOT = E // 128
MAXCH = CH_TOT // NW + 1


def _make_agg_global(D, nbuf, depth):
    ngrp = (MAXCH + nbuf - 1) // nbuf

    @functools.partial(
        pl.kernel,
        out_type=jax.ShapeDtypeStruct((NC, N, 128), jnp.float32),
        mesh=_MESH,
        scratch_types=[
            pltpu.VMEM((MAXCH, 128), jnp.int32),
            pltpu.VMEM((MAXCH, 128), jnp.int32),
            pltpu.VMEM((nbuf, 128, D), jnp.float32),
            pltpu.VMEM_SHARED((N, D), jnp.float32),
            [pltpu.SemaphoreType.DMA] * nbuf,
            [pltpu.SemaphoreType.DMA] * nbuf,
        ],
        compiler_params=_SC_PARAMS,
        name=f"gcn_agg_{D}",
    )
    def agg(y_hbm, src_hbm, dst_hbm, zeros_hbm, out_hbm,
            sidx, didx, bufs, acc, gsem, ssem):
        cid = lax.axis_index("c")
        sid = lax.axis_index("s")
        wid = cid * NS + sid
        rw = CH_TOT * wid // NW
        cnt = CH_TOT * (wid + 1) // NW - rw
        pltpu.sync_copy(src_hbm.at[pl.ds(rw, MAXCH)], sidx)
        pltpu.sync_copy(dst_hbm.at[pl.ds(rw, MAXCH)], didx)
        for b in range(depth):
            pltpu.async_copy(y_hbm.at[sidx.at[b]], bufs.at[b], gsem[b])
        pltpu.sync_copy(zeros_hbm, acc.at[pl.ds(sid * RPT, RPT)])
        plsc.subcore_barrier()

        def body(g, carry):
            for b in range(nbuf):
                j = g * nbuf + b
                jg = j + depth
                bg = (b + depth) % nbuf

                @pl.when((jg >= nbuf) & (jg < cnt))
                def _():
                    pltpu.make_async_copy(bufs.at[bg], acc.at[didx.at[0]],
                                          ssem[bg]).wait()

                @pl.when(jg < cnt)
                def _():
                    pltpu.async_copy(y_hbm.at[sidx.at[jg]], bufs.at[bg],
                                     gsem[bg])

                @pl.when(j < cnt)
                def _():
                    pltpu.make_async_copy(y_hbm.at[sidx.at[j]], bufs.at[b],
                                          gsem[b]).wait()
                    pltpu.async_copy(bufs.at[b], acc.at[didx.at[j]], ssem[b],
                                     add=True)
            return carry

        lax.fori_loop(0, ngrp, body, 0)
        for b in range(nbuf):
            pltpu.make_async_copy(bufs.at[b], acc.at[didx.at[0]],
                                  ssem[b]).wait()
        plsc.subcore_barrier()
        pltpu.sync_copy(acc.at[pl.ds(sid * RPT, RPT)],
                        out_hbm.at[cid, pl.ds(sid * RPT, RPT), pl.ds(0, D)])

    return agg


_agg64 = _make_agg_global(H2, nbuf=8, depth=5)
_agg16 = _make_agg_global(C, nbuf=10, depth=6)


BM = 2000


def _esplit_body(e_ref, s_ref, d_ref):
    s_ref[...] = e_ref[0:1, :].reshape(E // 128, 128)
    d_ref[...] = e_ref[1:2, :].reshape(E // 128, 128)


def _esplit(edge_index):
    return pl.pallas_call(
        _esplit_body,
        grid=(1,),
        in_specs=[pl.BlockSpec((2, E), lambda i: (0, 0))],
        out_specs=[
            pl.BlockSpec((E // 128, 128), lambda i: (0, 0)),
            pl.BlockSpec((E // 128, 128), lambda i: (0, 0)),
        ],
        out_shape=[
            jax.ShapeDtypeStruct((E // 128, 128), jnp.int32),
            jax.ShapeDtypeStruct((E // 128, 128), jnp.int32),
        ],
    )(edge_index)


def _dinv_of(dp_ref):
    deg = dp_ref[0, :, 0:1] + dp_ref[1, :, 0:1] + 1.0
    return lax.rsqrt(deg)


def _mm1_body(x_ref, w_ref, dp_ref, o_ref):
    dinv = _dinv_of(dp_ref)
    o_ref[...] = jnp.dot(x_ref[...], w_ref[...],
                         preferred_element_type=jnp.float32) * dinv


def _mm1(x, W, dp):
    return pl.pallas_call(
        _mm1_body,
        grid=(N // BM,),
        in_specs=[
            pl.BlockSpec((BM, D_IN), lambda i: (i, 0)),
            pl.BlockSpec((D_IN, H1), lambda i: (0, 0)),
            pl.BlockSpec((NC, BM, DEG_W), lambda i: (0, i, 0)),
        ],
        out_specs=pl.BlockSpec((BM, H1), lambda i: (i, 0)),
        out_shape=jax.ShapeDtypeStruct((N, H1), jnp.float32),
    )(x, W, dp)


def _make_mmf_body(din, dout, pack_out):
    def _mmf_body(z_ref, y_ref, b_ref, w_ref, dp_ref, o_ref, op_ref=None):
        dinv = _dinv_of(dp_ref)
        h = jnp.maximum(
            dinv * (z_ref[0, :, 0:din] + z_ref[1, :, 0:din] + y_ref[...])
            + b_ref[...], 0.0)
        o = jnp.dot(h, w_ref[...], preferred_element_type=jnp.float32) * dinv
        o_ref[...] = o
    return _mmf_body


def _mmf(z, y, b, W, dp, pack_out):
    din, dout = W.shape
    out_shape = [jax.ShapeDtypeStruct((N, dout), jnp.float32)]
    out_specs = [pl.BlockSpec((BM, dout), lambda i: (i, 0))]
    if pack_out:
        out_shape.append(
            jax.ShapeDtypeStruct((N * dout // 128, 128), jnp.float32))
        out_specs.append(
            pl.BlockSpec((BM * dout // 128, 128), lambda i: (i, 0)))
    return pl.pallas_call(
        _make_mmf_body(din, dout, pack_out),
        grid=(N // BM,),
        in_specs=[
            pl.BlockSpec((NC, BM, 128), lambda i: (0, i, 0)),
            pl.BlockSpec((BM, din), lambda i: (i, 0)),
            pl.BlockSpec((1, din), lambda i: (0, 0)),
            pl.BlockSpec((din, dout), lambda i: (0, 0)),
            pl.BlockSpec((NC, BM, DEG_W), lambda i: (0, i, 0)),
        ],
        out_specs=out_specs,
        out_shape=out_shape,
    )(z, y, b, W, dp)


def _sm_body(z_ref, y_ref, b_ref, dp_ref, o_ref):
    dinv = _dinv_of(dp_ref)
    logits = (dinv * (z_ref[0, :, 0:C] + z_ref[1, :, 0:C] + y_ref[...])
              + b_ref[...])
    m = jnp.max(logits, axis=1, keepdims=True)
    e = jnp.exp(logits - m)
    o_ref[...] = e / jnp.sum(e, axis=1, keepdims=True)


def _sm(z, y, b, dp):
    return pl.pallas_call(
        _sm_body,
        grid=(N // BM,),
        in_specs=[
            pl.BlockSpec((NC, BM, 128), lambda i: (0, i, 0)),
            pl.BlockSpec((BM, C), lambda i: (i, 0)),
            pl.BlockSpec((1, C), lambda i: (0, 0)),
            pl.BlockSpec((NC, BM, DEG_W), lambda i: (0, i, 0)),
        ],
        out_specs=pl.BlockSpec((BM, C), lambda i: (i, 0)),
        out_shape=jax.ShapeDtypeStruct((N, C), jnp.float32),
    )(z, y, b, dp)


def kernel(x, edge_index, W1, b1, W2, b2, W3, b3):
    srcp, dstp = _esplit(edge_index)
    src = srcp.reshape(NW, EPW)
    dst = dstp.reshape(NW, EPW)

    dp = _deg(dst.reshape(NW, EPW // DEG_K, DEG_K),
              jnp.zeros((RPT, DEG_W), jnp.float32),
              jnp.ones((DEG_K, DEG_W), jnp.float32))

    y1 = _mm1(x, W1, dp)
    z1 = _agg128(y1, src, dst.reshape(NW, EPW // 80, 80),
                 jnp.zeros((RPT, H1), jnp.float32))
    y2, = _mmf(z1, y1, b1.reshape(1, H1), W2, dp, pack_out=False)
    z2 = _agg64(y2, srcp, dstp, jnp.zeros((RPT, H2), jnp.float32))
    y3, = _mmf(z2, y2, b2.reshape(1, H2), W3, dp, pack_out=False)
    z3 = _agg16(y3, srcp, dstp, jnp.zeros((RPT, C), jnp.float32))
    return _sm(z3, y3, b3.reshape(1, C), dp)


# speedup vs baseline: 45.6094x; 1.0022x over previous
import functools

import jax
import jax.numpy as jnp
from jax import lax
from jax.experimental import pallas as pl
from jax.experimental.pallas import tpu as pltpu
from jax.experimental.pallas import tpu_sc as plsc

N = 10000
E = 320000
D_IN = 128
H1 = 128
H2 = 64
C = 16

NC = 2
NS = 16
NW = NC * NS
EPW = E // NW
RPT = N // NS
DEG_W = 8
DEG_K = 80

_MESH = plsc.VectorSubcoreMesh(core_axis_name="c", subcore_axis_name="s")
_SC_PARAMS = pltpu.CompilerParams(use_tc_tiling_on_sc=False)


CH_TOT = E // 128
MAXCH = CH_TOT // NW + 1


@functools.partial(
    pl.kernel,
    out_type=jax.ShapeDtypeStruct((NC, N, DEG_W), jnp.float32),
    mesh=_MESH,
    scratch_types=[
        pltpu.VMEM((MAXCH, 128), jnp.int32),
        pltpu.VMEM((128, DEG_W), jnp.float32),
        pltpu.VMEM_SHARED((N, DEG_W), jnp.float32),
        pltpu.SemaphoreType.DMA,
    ],
    compiler_params=_SC_PARAMS,
)
def _deg(dst_hbm, zeros_hbm, ones_hbm, out_hbm, didx, onesb, acc, dsem):
    cid = lax.axis_index("c")
    sid = lax.axis_index("s")
    wid = cid * NS + sid
    rw = CH_TOT * wid // NW
    cnt = CH_TOT * (wid + 1) // NW - rw
    pltpu.sync_copy(zeros_hbm, acc.at[pl.ds(sid * RPT, RPT)])
    pltpu.sync_copy(ones_hbm, onesb)
    pltpu.sync_copy(dst_hbm.at[pl.ds(rw, MAXCH)], didx)
    plsc.subcore_barrier()

    def body(j, carry):
        pltpu.async_copy(onesb, acc.at[didx.at[j]], dsem, add=True)
        return carry

    lax.fori_loop(0, cnt, body, 0)

    def drain(j, carry):
        pltpu.make_async_copy(onesb, acc.at[didx.at[0]], dsem).wait()
        return carry

    lax.fori_loop(0, cnt, drain, 0)
    plsc.subcore_barrier()
    pltpu.sync_copy(acc.at[pl.ds(sid * RPT, RPT)],
                    out_hbm.at[cid, pl.ds(sid * RPT, RPT)])


def _make_agg(D, K, nbuf, depth):
    nch = EPW // K
    ngrp = (nch + nbuf - 1) // nbuf

    @functools.partial(
        pl.kernel,
        out_type=jax.ShapeDtypeStruct((NC, N, 128), jnp.float32),
        mesh=_MESH,
        scratch_types=[
            pltpu.VMEM((EPW,), jnp.int32),
            pltpu.VMEM((nch, K), jnp.int32),
            pltpu.VMEM((nbuf, K, D), jnp.float32),
            pltpu.VMEM_SHARED((N, D), jnp.float32),
            [pltpu.SemaphoreType.DMA] * nbuf,
            [pltpu.SemaphoreType.DMA] * nbuf,
        ],
        compiler_params=_SC_PARAMS,
        name=f"gcn_agg_{D}",
    )
    def agg(y_hbm, src_hbm, dst_hbm, zeros_hbm, out_hbm,
            sidx, didx, bufs, acc, gsem, ssem):
        cid = lax.axis_index("c")
        sid = lax.axis_index("s")
        wid = cid * NS + sid
        pltpu.sync_copy(src_hbm.at[wid], sidx)
        pltpu.sync_copy(dst_hbm.at[wid], didx)
        for b in range(depth):
            pltpu.async_copy(y_hbm.at[sidx.at[pl.ds(b * K, K)]],
                             bufs.at[b], gsem[b])
        pltpu.sync_copy(zeros_hbm, acc.at[pl.ds(sid * RPT, RPT)])
        plsc.subcore_barrier()

        def body(g, carry):
            for b in range(nbuf):
                j = g * nbuf + b
                jg = j + depth
                bg = (b + depth) % nbuf

                @pl.when((jg >= nbuf) & (jg < nch))
                def _():
                    pltpu.make_async_copy(bufs.at[bg], acc.at[didx.at[0]],
                                          ssem[bg]).wait()

                @pl.when(jg < nch)
                def _():
                    pltpu.async_copy(y_hbm.at[sidx.at[pl.ds(jg * K, K)]],
                                     bufs.at[bg], gsem[bg])

                @pl.when(j < nch)
                def _():
                    pltpu.make_async_copy(
                        y_hbm.at[sidx.at[pl.ds(j * K, K)]], bufs.at[b],
                        gsem[b]).wait()
                    pltpu.async_copy(bufs.at[b], acc.at[didx.at[j]], ssem[b],
                                     add=True)
            return carry

        lax.fori_loop(0, ngrp, body, 0)
        for b in range(nbuf):
            pltpu.make_async_copy(bufs.at[b], acc.at[didx.at[0]],
                                  ssem[b]).wait()
        plsc.subcore_barrier()
        pltpu.sync_copy(acc.at[pl.ds(sid * RPT, RPT)],
                        out_hbm.at[cid, pl.ds(sid * RPT, RPT), pl.ds(0, D)])

    return agg


_agg128 = _make_agg(H1, K=80, nbuf=3, depth=2)

def _make_agg_global(D, nbuf, depth):
    ngrp = (MAXCH + nbuf - 1) // nbuf

    @functools.partial(
        pl.kernel,
        out_type=jax.ShapeDtypeStruct((NC, N, 128), jnp.float32),
        mesh=_MESH,
        scratch_types=[
            pltpu.VMEM((MAXCH, 128), jnp.int32),
            pltpu.VMEM((MAXCH, 128), jnp.int32),
            pltpu.VMEM((nbuf, 128, D), jnp.float32),
            pltpu.VMEM_SHARED((N, D), jnp.float32),
            [pltpu.SemaphoreType.DMA] * nbuf,
            [pltpu.SemaphoreType.DMA] * nbuf,
        ],
        compiler_params=_SC_PARAMS,
        name=f"gcn_agg_{D}",
    )
    def agg(y_hbm, src_hbm, dst_hbm, zeros_hbm, out_hbm,
            sidx, didx, bufs, acc, gsem, ssem):
        cid = lax.axis_index("c")
        sid = lax.axis_index("s")
        wid = cid * NS + sid
        rw = CH_TOT * wid // NW
        cnt = CH_TOT * (wid + 1) // NW - rw
        pltpu.sync_copy(src_hbm.at[pl.ds(rw, MAXCH)], sidx)
        pltpu.sync_copy(dst_hbm.at[pl.ds(rw, MAXCH)], didx)
        for b in range(depth):
            pltpu.async_copy(y_hbm.at[sidx.at[b]], bufs.at[b], gsem[b])
        pltpu.sync_copy(zeros_hbm, acc.at[pl.ds(sid * RPT, RPT)])
        plsc.subcore_barrier()

        def body(g, carry):
            for b in range(nbuf):
                j = g * nbuf + b
                jg = j + depth
                bg = (b + depth) % nbuf

                @pl.when((jg >= nbuf) & (jg < cnt))
                def _():
                    pltpu.make_async_copy(bufs.at[bg], acc.at[didx.at[0]],
                                          ssem[bg]).wait()

                @pl.when(jg < cnt)
                def _():
                    pltpu.async_copy(y_hbm.at[sidx.at[jg]], bufs.at[bg],
                                     gsem[bg])

                @pl.when(j < cnt)
                def _():
                    pltpu.make_async_copy(y_hbm.at[sidx.at[j]], bufs.at[b],
                                          gsem[b]).wait()
                    pltpu.async_copy(bufs.at[b], acc.at[didx.at[j]], ssem[b],
                                     add=True)
            return carry

        lax.fori_loop(0, ngrp, body, 0)
        for b in range(nbuf):
            pltpu.make_async_copy(bufs.at[b], acc.at[didx.at[0]],
                                  ssem[b]).wait()
        plsc.subcore_barrier()
        pltpu.sync_copy(acc.at[pl.ds(sid * RPT, RPT)],
                        out_hbm.at[cid, pl.ds(sid * RPT, RPT), pl.ds(0, D)])

    return agg


_agg64 = _make_agg_global(H2, nbuf=8, depth=5)
_agg16 = _make_agg_global(C, nbuf=14, depth=10)


BM = 2000


def _esplit_body(e_ref, s_ref, d_ref):
    s_ref[...] = e_ref[0:1, :].reshape(E // 128, 128)
    d_ref[...] = e_ref[1:2, :].reshape(E // 128, 128)


def _esplit(edge_index):
    return pl.pallas_call(
        _esplit_body,
        grid=(1,),
        in_specs=[pl.BlockSpec((2, E), lambda i: (0, 0))],
        out_specs=[
            pl.BlockSpec((E // 128, 128), lambda i: (0, 0)),
            pl.BlockSpec((E // 128, 128), lambda i: (0, 0)),
        ],
        out_shape=[
            jax.ShapeDtypeStruct((E // 128, 128), jnp.int32),
            jax.ShapeDtypeStruct((E // 128, 128), jnp.int32),
        ],
    )(edge_index)


def _dinv_of(dp_ref):
    deg = dp_ref[0, :, 0:1] + dp_ref[1, :, 0:1] + 1.0
    return lax.rsqrt(deg)


def _mm1_body(x_ref, w_ref, dp_ref, o_ref):
    dinv = _dinv_of(dp_ref)
    o_ref[...] = jnp.dot(x_ref[...], w_ref[...],
                         preferred_element_type=jnp.float32) * dinv


def _mm1(x, W, dp):
    return pl.pallas_call(
        _mm1_body,
        grid=(N // BM,),
        in_specs=[
            pl.BlockSpec((BM, D_IN), lambda i: (i, 0)),
            pl.BlockSpec((D_IN, H1), lambda i: (0, 0)),
            pl.BlockSpec((NC, BM, DEG_W), lambda i: (0, i, 0)),
        ],
        out_specs=pl.BlockSpec((BM, H1), lambda i: (i, 0)),
        out_shape=jax.ShapeDtypeStruct((N, H1), jnp.float32),
    )(x, W, dp)


def _make_mmf_body(din, dout, pack_out):
    def _mmf_body(z_ref, y_ref, b_ref, w_ref, dp_ref, o_ref, op_ref=None):
        dinv = _dinv_of(dp_ref)
        h = jnp.maximum(
            dinv * (z_ref[0, :, 0:din] + z_ref[1, :, 0:din] + y_ref[...])
            + b_ref[...], 0.0)
        o = jnp.dot(h, w_ref[...], preferred_element_type=jnp.float32) * dinv
        o_ref[...] = o
    return _mmf_body


def _mmf(z, y, b, W, dp, pack_out):
    din, dout = W.shape
    out_shape = [jax.ShapeDtypeStruct((N, dout), jnp.float32)]
    out_specs = [pl.BlockSpec((BM, dout), lambda i: (i, 0))]
    if pack_out:
        out_shape.append(
            jax.ShapeDtypeStruct((N * dout // 128, 128), jnp.float32))
        out_specs.append(
            pl.BlockSpec((BM * dout // 128, 128), lambda i: (i, 0)))
    return pl.pallas_call(
        _make_mmf_body(din, dout, pack_out),
        grid=(N // BM,),
        in_specs=[
            pl.BlockSpec((NC, BM, 128), lambda i: (0, i, 0)),
            pl.BlockSpec((BM, din), lambda i: (i, 0)),
            pl.BlockSpec((1, din), lambda i: (0, 0)),
            pl.BlockSpec((din, dout), lambda i: (0, 0)),
            pl.BlockSpec((NC, BM, DEG_W), lambda i: (0, i, 0)),
        ],
        out_specs=out_specs,
        out_shape=out_shape,
    )(z, y, b, W, dp)


def _sm_body(z_ref, y_ref, b_ref, dp_ref, o_ref):
    dinv = _dinv_of(dp_ref)
    logits = (dinv * (z_ref[0, :, 0:C] + z_ref[1, :, 0:C] + y_ref[...])
              + b_ref[...])
    m = jnp.max(logits, axis=1, keepdims=True)
    e = jnp.exp(logits - m)
    o_ref[...] = e / jnp.sum(e, axis=1, keepdims=True)


def _sm(z, y, b, dp):
    return pl.pallas_call(
        _sm_body,
        grid=(N // BM,),
        in_specs=[
            pl.BlockSpec((NC, BM, 128), lambda i: (0, i, 0)),
            pl.BlockSpec((BM, C), lambda i: (i, 0)),
            pl.BlockSpec((1, C), lambda i: (0, 0)),
            pl.BlockSpec((NC, BM, DEG_W), lambda i: (0, i, 0)),
        ],
        out_specs=pl.BlockSpec((BM, C), lambda i: (i, 0)),
        out_shape=jax.ShapeDtypeStruct((N, C), jnp.float32),
    )(z, y, b, dp)


def kernel(x, edge_index, W1, b1, W2, b2, W3, b3):
    srcp, dstp = _esplit(edge_index)
    src = srcp.reshape(NW, EPW)
    dst = dstp.reshape(NW, EPW)

    dp = _deg(dstp,
              jnp.zeros((RPT, DEG_W), jnp.float32),
              jnp.ones((128, DEG_W), jnp.float32))

    y1 = _mm1(x, W1, dp)
    z1 = _agg128(y1, src, dst.reshape(NW, EPW // 80, 80),
                 jnp.zeros((RPT, H1), jnp.float32))
    y2, = _mmf(z1, y1, b1.reshape(1, H1), W2, dp, pack_out=False)
    z2 = _agg64(y2, srcp, dstp, jnp.zeros((RPT, H2), jnp.float32))
    y3, = _mmf(z2, y2, b2.reshape(1, H2), W3, dp, pack_out=False)
    z3 = _agg16(y3, srcp, dstp, jnp.zeros((RPT, C), jnp.float32))
    return _sm(z3, y3, b3.reshape(1, C), dp)


# speedup vs baseline: 45.7003x; 1.0020x over previous
import functools

import jax
import jax.numpy as jnp
from jax import lax
from jax.experimental import pallas as pl
from jax.experimental.pallas import tpu as pltpu
from jax.experimental.pallas import tpu_sc as plsc

N = 10000
E = 320000
D_IN = 128
H1 = 128
H2 = 64
C = 16

NC = 2
NS = 16
NW = NC * NS
EPW = E // NW
RPT = N // NS
DEG_W = 8

_MESH = plsc.VectorSubcoreMesh(core_axis_name="c", subcore_axis_name="s")
_SC_PARAMS = pltpu.CompilerParams(use_tc_tiling_on_sc=False)


CH_TOT = E // 128
MAXCH = CH_TOT // NW + 1


@functools.partial(
    pl.kernel,
    out_type=jax.ShapeDtypeStruct((NC, N, DEG_W), jnp.float32),
    mesh=_MESH,
    scratch_types=[
        pltpu.VMEM((MAXCH, 128), jnp.int32),
        pltpu.VMEM((128, DEG_W), jnp.float32),
        pltpu.VMEM_SHARED((N, DEG_W), jnp.float32),
        pltpu.SemaphoreType.DMA,
    ],
    compiler_params=_SC_PARAMS,
)
def _deg(dst_hbm, zeros_hbm, ones_hbm, out_hbm, didx, onesb, acc, dsem):
    cid = lax.axis_index("c")
    sid = lax.axis_index("s")
    wid = cid * NS + sid
    rw = CH_TOT * wid // NW
    cnt = CH_TOT * (wid + 1) // NW - rw
    pltpu.sync_copy(zeros_hbm, acc.at[pl.ds(sid * RPT, RPT)])
    pltpu.sync_copy(ones_hbm, onesb)
    pltpu.sync_copy(dst_hbm.at[pl.ds(rw, MAXCH)], didx)
    plsc.subcore_barrier()

    def body(j, carry):
        pltpu.async_copy(onesb, acc.at[didx.at[j]], dsem, add=True)
        return carry

    lax.fori_loop(0, cnt, body, 0)

    def drain(j, carry):
        pltpu.make_async_copy(onesb, acc.at[didx.at[0]], dsem).wait()
        return carry

    lax.fori_loop(0, cnt, drain, 0)
    plsc.subcore_barrier()
    pltpu.sync_copy(acc.at[pl.ds(sid * RPT, RPT)],
                    out_hbm.at[cid, pl.ds(sid * RPT, RPT)])


def _make_agg(D, K, nbuf, depth):
    nch = EPW // K
    ngrp = (nch + nbuf - 1) // nbuf

    @functools.partial(
        pl.kernel,
        out_type=jax.ShapeDtypeStruct((NC, N, 128), jnp.float32),
        mesh=_MESH,
        scratch_types=[
            pltpu.VMEM((EPW,), jnp.int32),
            pltpu.VMEM((nch, K), jnp.int32),
            pltpu.VMEM((nbuf, K, D), jnp.float32),
            pltpu.VMEM_SHARED((N, D), jnp.float32),
            [pltpu.SemaphoreType.DMA] * nbuf,
            [pltpu.SemaphoreType.DMA] * nbuf,
        ],
        compiler_params=_SC_PARAMS,
        name=f"gcn_agg_{D}",
    )
    def agg(y_hbm, src_hbm, dst_hbm, zeros_hbm, out_hbm,
            sidx, didx, bufs, acc, gsem, ssem):
        cid = lax.axis_index("c")
        sid = lax.axis_index("s")
        wid = cid * NS + sid
        pltpu.sync_copy(src_hbm.at[wid], sidx)
        pltpu.sync_copy(dst_hbm.at[wid], didx)
        for b in range(depth):
            pltpu.async_copy(y_hbm.at[sidx.at[pl.ds(b * K, K)]],
                             bufs.at[b], gsem[b])
        pltpu.sync_copy(zeros_hbm, acc.at[pl.ds(sid * RPT, RPT)])
        plsc.subcore_barrier()

        def body(g, carry):
            for b in range(nbuf):
                j = g * nbuf + b
                jg = j + depth
                bg = (b + depth) % nbuf

                @pl.when((jg >= nbuf) & (jg < nch))
                def _():
                    pltpu.make_async_copy(bufs.at[bg], acc.at[didx.at[0]],
                                          ssem[bg]).wait()

                @pl.when(jg < nch)
                def _():
                    pltpu.async_copy(y_hbm.at[sidx.at[pl.ds(jg * K, K)]],
                                     bufs.at[bg], gsem[bg])

                @pl.when(j < nch)
                def _():
                    pltpu.make_async_copy(
                        y_hbm.at[sidx.at[pl.ds(j * K, K)]], bufs.at[b],
                        gsem[b]).wait()
                    pltpu.async_copy(bufs.at[b], acc.at[didx.at[j]], ssem[b],
                                     add=True)
            return carry

        lax.fori_loop(0, ngrp, body, 0)
        for b in range(nbuf):
            pltpu.make_async_copy(bufs.at[b], acc.at[didx.at[0]],
                                  ssem[b]).wait()
        plsc.subcore_barrier()
        pltpu.sync_copy(acc.at[pl.ds(sid * RPT, RPT)],
                        out_hbm.at[cid, pl.ds(sid * RPT, RPT), pl.ds(0, D)])

    return agg


_agg128 = _make_agg(H1, K=80, nbuf=3, depth=2)

def _make_agg_global(D, nbuf, depth):
    ngrp = (MAXCH + nbuf - 1) // nbuf

    @functools.partial(
        pl.kernel,
        out_type=jax.ShapeDtypeStruct((NC, N, 128), jnp.float32),
        mesh=_MESH,
        scratch_types=[
            pltpu.VMEM((MAXCH, 128), jnp.int32),
            pltpu.VMEM((MAXCH, 128), jnp.int32),
            pltpu.VMEM((nbuf, 128, D), jnp.float32),
            pltpu.VMEM_SHARED((N, D), jnp.float32),
            [pltpu.SemaphoreType.DMA] * nbuf,
            [pltpu.SemaphoreType.DMA] * nbuf,
        ],
        compiler_params=_SC_PARAMS,
        name=f"gcn_agg_{D}",
    )
    def agg(y_hbm, src_hbm, dst_hbm, zeros_hbm, out_hbm,
            sidx, didx, bufs, acc, gsem, ssem):
        cid = lax.axis_index("c")
        sid = lax.axis_index("s")
        wid = cid * NS + sid
        rw = CH_TOT * wid // NW
        cnt = CH_TOT * (wid + 1) // NW - rw
        pltpu.sync_copy(src_hbm.at[pl.ds(rw, MAXCH)], sidx)
        pltpu.sync_copy(dst_hbm.at[pl.ds(rw, MAXCH)], didx)
        for b in range(depth):
            pltpu.async_copy(y_hbm.at[sidx.at[b]], bufs.at[b], gsem[b])
        pltpu.sync_copy(zeros_hbm, acc.at[pl.ds(sid * RPT, RPT)])
        plsc.subcore_barrier()

        def body(g, carry):
            for b in range(nbuf):
                j = g * nbuf + b
                jg = j + depth
                bg = (b + depth) % nbuf

                @pl.when((jg >= nbuf) & (jg < cnt))
                def _():
                    pltpu.make_async_copy(bufs.at[bg], acc.at[didx.at[0]],
                                          ssem[bg]).wait()

                @pl.when(jg < cnt)
                def _():
                    pltpu.async_copy(y_hbm.at[sidx.at[jg]], bufs.at[bg],
                                     gsem[bg])

                @pl.when(j < cnt)
                def _():
                    pltpu.make_async_copy(y_hbm.at[sidx.at[j]], bufs.at[b],
                                          gsem[b]).wait()
                    pltpu.async_copy(bufs.at[b], acc.at[didx.at[j]], ssem[b],
                                     add=True)
            return carry

        lax.fori_loop(0, ngrp, body, 0)
        for b in range(nbuf):
            pltpu.make_async_copy(bufs.at[b], acc.at[didx.at[0]],
                                  ssem[b]).wait()
        plsc.subcore_barrier()
        pltpu.sync_copy(acc.at[pl.ds(sid * RPT, RPT)],
                        out_hbm.at[cid, pl.ds(sid * RPT, RPT), pl.ds(0, D)])

    return agg


_agg64 = _make_agg_global(H2, nbuf=8, depth=5)
_agg16 = _make_agg_global(C, nbuf=14, depth=10)


BM = 2000


def _esplit_body(e_ref, s_ref, d_ref):
    s_ref[...] = e_ref[0:1, :].reshape(E // 128, 128)
    d_ref[...] = e_ref[1:2, :].reshape(E // 128, 128)


def _esplit(edge_index):
    return pl.pallas_call(
        _esplit_body,
        grid=(1,),
        in_specs=[pl.BlockSpec((2, E), lambda i: (0, 0))],
        out_specs=[
            pl.BlockSpec((E // 128, 128), lambda i: (0, 0)),
            pl.BlockSpec((E // 128, 128), lambda i: (0, 0)),
        ],
        out_shape=[
            jax.ShapeDtypeStruct((E // 128, 128), jnp.int32),
            jax.ShapeDtypeStruct((E // 128, 128), jnp.int32),
        ],
    )(edge_index)


def _dinv_of(dp_ref):
    deg = dp_ref[0, :, 0:1] + dp_ref[1, :, 0:1] + 1.0
    return lax.rsqrt(deg)


def _mm1_body(x_ref, w_ref, dp_ref, o_ref):
    dinv = _dinv_of(dp_ref)
    o_ref[...] = jnp.dot(x_ref[...], w_ref[...],
                         preferred_element_type=jnp.float32) * dinv


def _mm1(x, W, dp):
    return pl.pallas_call(
        _mm1_body,
        grid=(N // BM,),
        in_specs=[
            pl.BlockSpec((BM, D_IN), lambda i: (i, 0)),
            pl.BlockSpec((D_IN, H1), lambda i: (0, 0)),
            pl.BlockSpec((NC, BM, DEG_W), lambda i: (0, i, 0)),
        ],
        out_specs=pl.BlockSpec((BM, H1), lambda i: (i, 0)),
        out_shape=jax.ShapeDtypeStruct((N, H1), jnp.float32),
    )(x, W, dp)


def _make_mmf_body(din, dout, pack_out):
    def _mmf_body(z_ref, y_ref, b_ref, w_ref, dp_ref, o_ref, op_ref=None):
        dinv = _dinv_of(dp_ref)
        h = jnp.maximum(
            dinv * (z_ref[0, :, 0:din] + z_ref[1, :, 0:din] + y_ref[...])
            + b_ref[...], 0.0)
        o = jnp.dot(h, w_ref[...], preferred_element_type=jnp.float32) * dinv
        o_ref[...] = o
    return _mmf_body


def _mmf(z, y, b, W, dp, pack_out):
    din, dout = W.shape
    out_shape = [jax.ShapeDtypeStruct((N, dout), jnp.float32)]
    out_specs = [pl.BlockSpec((BM, dout), lambda i: (i, 0))]
    if pack_out:
        out_shape.append(
            jax.ShapeDtypeStruct((N * dout // 128, 128), jnp.float32))
        out_specs.append(
            pl.BlockSpec((BM * dout // 128, 128), lambda i: (i, 0)))
    return pl.pallas_call(
        _make_mmf_body(din, dout, pack_out),
        grid=(N // BM,),
        in_specs=[
            pl.BlockSpec((NC, BM, 128), lambda i: (0, i, 0)),
            pl.BlockSpec((BM, din), lambda i: (i, 0)),
            pl.BlockSpec((1, din), lambda i: (0, 0)),
            pl.BlockSpec((din, dout), lambda i: (0, 0)),
            pl.BlockSpec((NC, BM, DEG_W), lambda i: (0, i, 0)),
        ],
        out_specs=out_specs,
        out_shape=out_shape,
    )(z, y, b, W, dp)


def _sm_body(z_ref, y_ref, b_ref, dp_ref, o_ref):
    dinv = _dinv_of(dp_ref)
    logits = (dinv * (z_ref[0, :, 0:C] + z_ref[1, :, 0:C] + y_ref[...])
              + b_ref[...])
    m = jnp.max(logits, axis=1, keepdims=True)
    e = jnp.exp(logits - m)
    o_ref[...] = e / jnp.sum(e, axis=1, keepdims=True)


def _sm(z, y, b, dp):
    return pl.pallas_call(
        _sm_body,
        grid=(N // BM,),
        in_specs=[
            pl.BlockSpec((NC, BM, 128), lambda i: (0, i, 0)),
            pl.BlockSpec((BM, C), lambda i: (i, 0)),
            pl.BlockSpec((1, C), lambda i: (0, 0)),
            pl.BlockSpec((NC, BM, DEG_W), lambda i: (0, i, 0)),
        ],
        out_specs=pl.BlockSpec((BM, C), lambda i: (i, 0)),
        out_shape=jax.ShapeDtypeStruct((N, C), jnp.float32),
    )(z, y, b, dp)


def kernel(x, edge_index, W1, b1, W2, b2, W3, b3):
    srcp, dstp = _esplit(edge_index)
    src = srcp.reshape(NW, EPW)
    dst = dstp.reshape(NW, EPW)

    dp = _deg(dstp,
              jnp.zeros((RPT, DEG_W), jnp.float32),
              jnp.ones((128, DEG_W), jnp.float32))

    y1 = _mm1(x, W1, dp)
    z1 = _agg128(y1, src, dst.reshape(NW, EPW // 80, 80),
                 jnp.zeros((RPT, H1), jnp.float32))
    y2, = _mmf(z1, y1, b1.reshape(1, H1), W2, dp, pack_out=False)
    z2 = _agg64(y2, srcp, dstp, jnp.zeros((RPT, H2), jnp.float32))
    y3, = _mmf(z2, y2, b2.reshape(1, H2), W3, dp, pack_out=False)
    z3 = _agg16(y3, srcp, dstp, jnp.zeros((RPT, C), jnp.float32))
    return _sm(z3, y3, b3.reshape(1, C), dp)
